# baseline (speedup 1.0000x reference)
import functools

import jax
import jax.numpy as jnp
from jax import lax
from jax.experimental import pallas as pl
from jax.experimental.pallas import tpu as pltpu
from jax.experimental.pallas import tpu_sc as plsc

VOCAB = 1000000
EMB = 16
MAX_NORM = 100.0

NC, NS, L = 2, 16, 16
NW = NC * NS

B_TOTAL = 16384 * 50
PER_W = B_TOTAL // NW
G = 128
NGRP = PER_W // G


def _rsqrt(x):
    i = plsc.bitcast(x, jnp.int32)
    i = jnp.int32(0x5F3759DF) - lax.shift_right_arithmetic(i, 1)
    y = plsc.bitcast(i, jnp.float32)
    for _ in range(3):
        y = y * (1.5 - 0.5 * x * y * y)
    return y


def _body(table_hbm, idx_hbm, out_hbm, idx_v, rows_v, sem):
    wid = lax.axis_index("c") * NS + lax.axis_index("s")
    base = wid * PER_W

    pltpu.sync_copy(idx_hbm.at[wid], idx_v)

    riota = lax.iota(jnp.int32, L)

    def group(g, _):
        pltpu.async_copy(table_hbm.at[idx_v.at[g]], rows_v, sem).wait()

        def subgroup(s, _):
            row_idx = s * L + riota
            cols = []
            ss = jnp.zeros((L,), jnp.float32)
            for j in range(EMB):
                cj = plsc.load_gather(
                    rows_v, [row_idx, jnp.full((L,), j, jnp.int32)])
                cols.append(cj)
                ss = ss + cj * cj
            ssc = jnp.maximum(ss, 1e-14)
            scale = jnp.minimum(1.0, MAX_NORM * _rsqrt(ssc))
            for j in range(EMB):
                plsc.store_scatter(
                    rows_v, [row_idx, jnp.full((L,), j, jnp.int32)],
                    cols[j] * scale)
            return ()

        lax.fori_loop(0, G // L, subgroup, (), unroll=False)

        pltpu.sync_copy(rows_v, out_hbm.at[pl.ds(base + g * G, G)])
        return ()

    lax.fori_loop(0, NGRP, group, (), unroll=False)


@jax.jit
def _run(x_grp, W):
    k = pl.kernel(
        _body,
        out_type=jax.ShapeDtypeStruct((B_TOTAL, EMB), jnp.float32),
        mesh=plsc.VectorSubcoreMesh(core_axis_name="c", subcore_axis_name="s"),
        scratch_types=[
            pltpu.VMEM((NGRP, G), jnp.int32),
            pltpu.VMEM((G, EMB), jnp.float32),
            pltpu.SemaphoreType.DMA,
        ],
        compiler_params=pltpu.CompilerParams(
            needs_layout_passes=False, use_tc_tiling_on_sc=False),
    )
    return k(W, x_grp)


def kernel(x, W):
    Bseq, Lseq = x.shape
    x_grp = x.reshape(NW, NGRP, G).astype(jnp.int32)
    out = _run(x_grp, W)
    return out.reshape(Bseq, Lseq, EMB)


# speedup vs baseline: 1.1271x; 1.1271x over previous
import functools

import jax
import jax.numpy as jnp
from jax import lax
from jax.experimental import pallas as pl
from jax.experimental.pallas import tpu as pltpu
from jax.experimental.pallas import tpu_sc as plsc

VOCAB = 1000000
EMB = 16
MAX_NORM = 100.0

NC, NS, L = 2, 16, 16
NW = NC * NS

B_TOTAL = 16384 * 50
PER_W = B_TOTAL // NW
G = 128
NGRP = PER_W // G


def _rsqrt(x):
    i = plsc.bitcast(x, jnp.int32)
    i = jnp.int32(0x5F3759DF) - lax.shift_right_arithmetic(i, 1)
    y = plsc.bitcast(i, jnp.float32)
    for _ in range(3):
        y = y * (1.5 - 0.5 * x * y * y)
    return y


GPC = 4
CHUNK = G * GPC
NCHUNK = PER_W // CHUNK


def _body(table_hbm, idx_hbm, out_hbm, idx_v, rows0, rows1, gs0, gs1,
          ws0, ws1):
    wid = lax.axis_index("c") * NS + lax.axis_index("s")
    base = wid * PER_W

    pltpu.sync_copy(idx_hbm.at[wid], idx_v)

    riota = lax.iota(jnp.int32, L)
    bufs = (rows0, rows1)
    gsems = (gs0, gs1)
    wsems = (ws0, ws1)

    def fire_gather(c, b):
        for i in range(GPC):
            pltpu.async_copy(
                table_hbm.at[idx_v.at[c * GPC + i]],
                bufs[b].at[pl.ds(i * G, G)], gsems[b])

    def drain_gather(b):
        for i in range(GPC):
            pltpu.make_async_copy(
                table_hbm.at[pl.ds(0, G)],
                bufs[b].at[pl.ds(i * G, G)], gsems[b]).wait()

    def compute(b):
        rows_v = bufs[b]

        def subgroup(s, _):
            row_idx = s * L + riota
            cols = []
            ss = jnp.zeros((L,), jnp.float32)
            for j in range(EMB):
                cj = plsc.load_gather(
                    rows_v, [row_idx, jnp.full((L,), j, jnp.int32)])
                cols.append(cj)
                ss = ss + cj * cj
            ssc = jnp.maximum(ss, 1e-14)
            scale = jnp.minimum(1.0, MAX_NORM * _rsqrt(ssc))
            for j in range(EMB):
                plsc.store_scatter(
                    rows_v, [row_idx, jnp.full((L,), j, jnp.int32)],
                    cols[j] * scale)
            return ()

        lax.fori_loop(0, CHUNK // L, subgroup, (), unroll=False)

    fire_gather(0, 0)

    @pl.loop(0, NCHUNK, step=2)
    def chunk_pair(c0):
        for b in range(2):
            c = c0 + b
            drain_gather(b)

            @pl.when(c >= 1)
            def _():
                pltpu.make_async_copy(
                    bufs[1 - b], out_hbm.at[pl.ds(base, CHUNK)],
                    wsems[1 - b]).wait()

            @pl.when(c + 1 < NCHUNK)
            def _():
                fire_gather(c + 1, 1 - b)

            compute(b)
            pltpu.async_copy(
                bufs[b], out_hbm.at[pl.ds(base + c * CHUNK, CHUNK)],
                wsems[b])

    pltpu.make_async_copy(
        bufs[1], out_hbm.at[pl.ds(base, CHUNK)], wsems[1]).wait()


@jax.jit
def _run(x_grp, W):
    k = pl.kernel(
        _body,
        out_type=jax.ShapeDtypeStruct((B_TOTAL, EMB), jnp.float32),
        mesh=plsc.VectorSubcoreMesh(core_axis_name="c", subcore_axis_name="s"),
        scratch_types=[
            pltpu.VMEM((NGRP, G), jnp.int32),
            pltpu.VMEM((CHUNK, EMB), jnp.float32),
            pltpu.VMEM((CHUNK, EMB), jnp.float32),
            pltpu.SemaphoreType.DMA,
            pltpu.SemaphoreType.DMA,
            pltpu.SemaphoreType.DMA,
            pltpu.SemaphoreType.DMA,
        ],
        compiler_params=pltpu.CompilerParams(
            needs_layout_passes=False, use_tc_tiling_on_sc=False),
    )
    return k(W, x_grp)


def kernel(x, W):
    Bseq, Lseq = x.shape
    x_grp = x.reshape(NW, NGRP, G).astype(jnp.int32)
    out = _run(x_grp, W)
    return out.reshape(Bseq, Lseq, EMB)


# speedup vs baseline: 2.5533x; 2.2654x over previous
import jax
import jax.numpy as jnp
from jax import lax
from jax.experimental import pallas as pl
from jax.experimental.pallas import tpu as pltpu
from jax.experimental.pallas import tpu_sc as plsc

VOCAB = 1000000
EMB = 16
MAX_NORM = 100.0

NC, NS, L = 2, 16, 16
NW = NC * NS

SEQ = 50
NB = 16384
G = 128
TCW = NB // G // NW
CHUNK = TCW * G


def _rsqrt(x):
    i = plsc.bitcast(x, jnp.int32)
    i = jnp.int32(0x5F3759DF) - lax.shift_right_arithmetic(i, 1)
    y = plsc.bitcast(i, jnp.float32)
    for _ in range(3):
        y = y * (1.5 - 0.5 * x * y * y)
    return y


def _body(table_hbm, idx_hbm, out_hbm, idx_v, rows0, rows1, outt0, outt1,
          gs0, gs1, ws0, ws1):
    wid = lax.axis_index("c") * NS + lax.axis_index("s")

    pltpu.sync_copy(idx_hbm.at[:, pl.ds(wid * TCW, TCW)], idx_v)

    riota = lax.iota(jnp.int32, L)
    rows = (rows0, rows1)
    outt = (outt0, outt1)
    gsems = (gs0, gs1)
    wsems = (ws0, ws1)

    def fire_gather(s, b):
        for j in range(TCW):
            pltpu.async_copy(
                table_hbm.at[idx_v.at[s, j]],
                rows[b].at[pl.ds(j * G, G)], gsems[b])

    def drain_gather(b):
        for j in range(TCW):
            pltpu.make_async_copy(
                table_hbm.at[pl.ds(0, G)],
                rows[b].at[pl.ds(j * G, G)], gsems[b]).wait()

    def fire_write(s, b):
        for tr in range(2):
            pltpu.async_copy(
                outt[b].at[tr],
                out_hbm.at[s, pl.ds(tr * 128 + wid * TCW, TCW)], wsems[b])

    def drain_write(b):
        for tr in range(2):
            pltpu.make_async_copy(
                outt[b].at[tr],
                out_hbm.at[0, pl.ds(0, TCW)], wsems[b]).wait()

    def compute(b):
        rows_v = rows[b]
        outt_v = outt[b]

        def subgroup(sg, _):
            row_idx = sg * L + riota
            j = sg // (G // L)
            col = (sg % (G // L)) * L
            cols = []
            ss = jnp.zeros((L,), jnp.float32)
            for f in range(EMB):
                cf = plsc.load_gather(
                    rows_v, [row_idx, jnp.full((L,), f, jnp.int32)])
                cols.append(cf)
                ss = ss + cf * cf
            ssc = jnp.maximum(ss, 1e-14)
            scale = jnp.minimum(1.0, MAX_NORM * _rsqrt(ssc))
            for f in range(EMB):
                outt_v[f // 8, j, pl.ds((f % 8) * 128 + col, L)] = (
                    cols[f] * scale)
            return ()

        lax.fori_loop(0, CHUNK // L, subgroup, (), unroll=False)

    fire_gather(0, 0)

    @pl.loop(0, SEQ, step=2)
    def unit_pair(s0):
        for b in range(2):
            s = s0 + b
            drain_gather(b)

            @pl.when(s + 1 < SEQ)
            def _():
                fire_gather(s + 1, 1 - b)

            @pl.when(s >= 2)
            def _():
                drain_write(b)

            compute(b)
            fire_write(s, b)

    drain_write(0)
    drain_write(1)


@jax.jit
def _run(xt3, W):
    k = pl.kernel(
        _body,
        out_type=jax.ShapeDtypeStruct((SEQ, 256, 1024), jnp.float32),
        mesh=plsc.VectorSubcoreMesh(core_axis_name="c", subcore_axis_name="s"),
        scratch_types=[
            pltpu.VMEM((SEQ, TCW, G), jnp.int32),
            pltpu.VMEM((CHUNK, EMB), jnp.float32),
            pltpu.VMEM((CHUNK, EMB), jnp.float32),
            pltpu.VMEM((2, TCW, 1024), jnp.float32),
            pltpu.VMEM((2, TCW, 1024), jnp.float32),
            pltpu.SemaphoreType.DMA,
            pltpu.SemaphoreType.DMA,
            pltpu.SemaphoreType.DMA,
            pltpu.SemaphoreType.DMA,
        ],
        compiler_params=pltpu.CompilerParams(
            needs_layout_passes=False, use_tc_tiling_on_sc=False),
    )
    return k(W, xt3)


def kernel(x, W):
    xt3 = x.T.astype(jnp.int32).reshape(SEQ, NB // G, G)
    out = _run(xt3, W)
    out = out.reshape(SEQ, 2, 128, 8, 128)
    out = out.transpose(2, 4, 0, 1, 3)
    return out.reshape(NB, SEQ, EMB)


# speedup vs baseline: 5.7638x; 2.2574x over previous
import jax
import jax.numpy as jnp
from jax import lax
from jax.experimental import pallas as pl
from jax.experimental.pallas import tpu as pltpu
from jax.experimental.pallas import tpu_sc as plsc

VOCAB = 1000000
EMB = 16
MAX_NORM = 100.0

NC, NS, L = 2, 16, 16
NW = NC * NS

SEQ = 50
NB = 16384
G = 128
TCW = NB // G // NW
CHUNK = TCW * G

AU = 512
NFULL = (VOCAB // G) // (AU // G)
UW = 61
TAIL0 = NFULL * AU


def _rsqrt(x):
    i = plsc.bitcast(x, jnp.int32)
    i = jnp.int32(0x5F3759DF) - lax.shift_right_arithmetic(i, 1)
    y = plsc.bitcast(i, jnp.float32)
    for _ in range(3):
        y = y * (1.5 - 0.5 * x * y * y)
    return y


def _body_detile(wt_hbm, wtail_hbm, wlin_hbm, bin0, bin1, bout0, bout1,
                 gs0, gs1, ws0, ws1):
    wid = lax.axis_index("c") * NS + lax.axis_index("s")
    base_u = wid * UW

    riota = lax.iota(jnp.int32, L)
    bins = (bin0, bin1)
    bouts = (bout0, bout1)
    gsems = (gs0, gs1)
    wsems = (ws0, ws1)

    def fire_in(gu, b):
        pltpu.async_copy(
            wt_hbm.at[:, :, pl.ds(gu * AU, AU)], bins[b], gsems[b])

    def drain_in(b):
        pltpu.make_async_copy(
            wt_hbm.at[:, :, pl.ds(0, AU)], bins[b], gsems[b]).wait()

    def fire_write(gu, b):
        pltpu.async_copy(
            bouts[b], wlin_hbm.at[pl.ds(gu * (AU * EMB // 128), 64)],
            wsems[b])

    def drain_write(b):
        pltpu.make_async_copy(
            bouts[b], wlin_hbm.at[pl.ds(0, 64)], wsems[b]).wait()

    def transpose(b):
        bin_v = bins[b]
        bout_v = bouts[b]

        def subgroup(sg, _):
            row = sg * L + riota
            q = lax.shift_right_logical(row, 3)
            rm = lax.shift_left(jnp.bitwise_and(row, 7), 4)
            for f in range(EMB):
                cf = bin_v[f // 8, f % 8, pl.ds(sg * L, L)]
                plsc.store_scatter(bout_v, [q, rm + f], cf)
            return ()

        lax.fori_loop(0, AU // L, subgroup, (), unroll=False)

    fire_in(base_u, 0)

    @pl.loop(0, UW - 1, step=2)
    def unit_pair(u0):
        for b in range(2):
            u = u0 + b
            drain_in(b)
            fire_in(base_u + u + 1, 1 - b)

            @pl.when(u >= 2)
            def _():
                drain_write(b)

            transpose(b)
            fire_write(base_u + u, b)

    drain_in(0)
    drain_write(0)
    transpose(0)
    fire_write(base_u + UW - 1, 0)
    drain_write(1)
    drain_write(0)

    @pl.when(wid == NW - 1)
    def _():
        fire_in(NFULL - 1, 0)
        drain_in(0)
        transpose(0)
        fire_write(NFULL - 1, 0)
        drain_write(0)

        pltpu.sync_copy(wtail_hbm, bout0.at[pl.ds(0, 8)])
        pltpu.sync_copy(
            bout0.at[pl.ds(0, 8)],
            wlin_hbm.at[pl.ds(TAIL0 * EMB // 128, 8)])


def _body_lookup(table_hbm, idx_hbm, out_hbm, idx_v, rows0, rows1,
                 outt0, outt1, gs0, gs1, ws0, ws1):
    wid = lax.axis_index("c") * NS + lax.axis_index("s")

    pltpu.sync_copy(idx_hbm.at[:, pl.ds(wid * TCW, TCW)], idx_v)

    riota = lax.iota(jnp.int32, L)
    rows = (rows0, rows1)
    outt = (outt0, outt1)
    gsems = (gs0, gs1)
    wsems = (ws0, ws1)

    def fire_gather(s, b):
        for j in range(TCW):
            pltpu.async_copy(
                table_hbm.at[idx_v.at[s, j]],
                rows[b].at[pl.ds(j * G, G)], gsems[b])

    def drain_gather(b):
        for j in range(TCW):
            pltpu.make_async_copy(
                table_hbm.at[pl.ds(0, G)],
                rows[b].at[pl.ds(j * G, G)], gsems[b]).wait()

    def fire_write(s, b):
        for tr in range(2):
            pltpu.async_copy(
                outt[b].at[tr],
                out_hbm.at[s, pl.ds(tr * 128 + wid * TCW, TCW)], wsems[b])

    def drain_write(b):
        for tr in range(2):
            pltpu.make_async_copy(
                outt[b].at[tr],
                out_hbm.at[0, pl.ds(0, TCW)], wsems[b]).wait()

    def compute(b):
        rows_v = rows[b]
        outt_v = outt[b]

        def subgroup(sg, _):
            row_idx = sg * L + riota
            j = sg // (G // L)
            col = (sg % (G // L)) * L
            cols = []
            ss = jnp.zeros((L,), jnp.float32)
            for f in range(EMB):
                cf = plsc.load_gather(
                    rows_v, [row_idx, jnp.full((L,), f, jnp.int32)])
                cols.append(cf)
                ss = ss + cf * cf
            ssc = jnp.maximum(ss, 1e-14)
            scale = jnp.minimum(1.0, MAX_NORM * _rsqrt(ssc))
            for f in range(EMB):
                outt_v[f // 8, j, pl.ds((f % 8) * 128 + col, L)] = (
                    cols[f] * scale)
            return ()

        lax.fori_loop(0, CHUNK // L, subgroup, (), unroll=False)

    fire_gather(0, 0)

    @pl.loop(0, SEQ, step=2)
    def unit_pair(s0):
        for b in range(2):
            s = s0 + b
            drain_gather(b)

            @pl.when(s + 1 < SEQ)
            def _():
                fire_gather(s + 1, 1 - b)

            @pl.when(s >= 2)
            def _():
                drain_write(b)

            compute(b)
            fire_write(s, b)

    drain_write(0)
    drain_write(1)


@jax.jit
def _run(x, W):
    wt3 = W.T.reshape(2, 8, VOCAB)
    wtail = lax.slice(W, (TAIL0, 0), (VOCAB, EMB)).reshape(8, 128)
    xt3 = x.T.astype(jnp.int32).reshape(SEQ, NB // G, G)

    detile = pl.kernel(
        _body_detile,
        out_type=jax.ShapeDtypeStruct((VOCAB * EMB // 128, 128),
                                      jnp.float32),
        mesh=plsc.VectorSubcoreMesh(core_axis_name="c", subcore_axis_name="s"),
        scratch_types=[
            pltpu.VMEM((2, 8, AU), jnp.float32),
            pltpu.VMEM((2, 8, AU), jnp.float32),
            pltpu.VMEM((64, 128), jnp.float32),
            pltpu.VMEM((64, 128), jnp.float32),
            pltpu.SemaphoreType.DMA,
            pltpu.SemaphoreType.DMA,
            pltpu.SemaphoreType.DMA,
            pltpu.SemaphoreType.DMA,
        ],
        compiler_params=pltpu.CompilerParams(
            needs_layout_passes=False, use_tc_tiling_on_sc=True),
    )
    w_rm = detile(wt3, wtail).reshape(VOCAB, EMB)

    lookup = pl.kernel(
        _body_lookup,
        out_type=jax.ShapeDtypeStruct((SEQ, 256, 1024), jnp.float32),
        mesh=plsc.VectorSubcoreMesh(core_axis_name="c", subcore_axis_name="s"),
        scratch_types=[
            pltpu.VMEM((SEQ, TCW, G), jnp.int32),
            pltpu.VMEM((CHUNK, EMB), jnp.float32),
            pltpu.VMEM((CHUNK, EMB), jnp.float32),
            pltpu.VMEM((2, TCW, 1024), jnp.float32),
            pltpu.VMEM((2, TCW, 1024), jnp.float32),
            pltpu.SemaphoreType.DMA,
            pltpu.SemaphoreType.DMA,
            pltpu.SemaphoreType.DMA,
            pltpu.SemaphoreType.DMA,
        ],
        compiler_params=pltpu.CompilerParams(
            needs_layout_passes=False, use_tc_tiling_on_sc=False),
    )
    out = lookup(w_rm, xt3)

    out = out.reshape(SEQ, 2, 128, 8, 128)
    out = out.transpose(2, 4, 0, 1, 3)
    return out.reshape(NB, SEQ, EMB)


def kernel(x, W):
    return _run(x, W)
